# baseline (speedup 1.0000x reference)
import functools

import jax
import jax.numpy as jnp
from jax import lax
from jax.experimental import pallas as pl
from jax.experimental.pallas import tpu as pltpu
from jax.experimental.pallas import tpu_sc as plsc

NF = 8
VOCAB = 100000
BATCH = 16384
EMBED = 64
LANES = 16

NC = 2
NS = 16
NW = NC * NS

RCHUNK = 80
NCHUNKS = (VOCAB // 2) // RCHUNK
KMAX = -(-NCHUNKS // NW)

BPW = BATCH // NW
CHUNK = 64
ROUNDS = BPW // CHUNK
GROUPS = CHUNK // LANES


def _relayout_body(w0, w1, w2, w3, w4, w5, w6, w7,
                   o0, o1, o2, o3, o4, o5, o6, o7,
                   bufA0, bufA1, bufB0, bufB1,
                   semA0, semA1, semB0, semB1):
    tables = [w0, w1, w2, w3, w4, w5, w6, w7]
    outs = [o0, o1, o2, o3, o4, o5, o6, o7]
    bufA = [bufA0, bufA1]
    bufB = [bufB0, bufB1]
    semA = [semA0, semA1]
    semB = [semB0, semB1]
    wid = lax.axis_index("s") * NC + lax.axis_index("c")

    for t in range(NF):
        w, o = tables[t], outs[t]

        def issue_in(k, b):
            c = wid + k * NW

            @pl.when(c < NCHUNKS)
            def _():
                pltpu.async_copy(w.at[pl.ds(c * 2 * RCHUNK, 2 * RCHUNK)],
                                 bufA[b], semA[b])

        def wait_in(k, b):
            c = wid + k * NW

            @pl.when(c < NCHUNKS)
            def _():
                pltpu.make_async_copy(w.at[pl.ds(0, 2 * RCHUNK)],
                                      bufA[b], semA[b]).wait()

        def issue_out(k, b):
            c = wid + k * NW

            @pl.when(c < NCHUNKS)
            def _():
                pltpu.async_copy(bufB[b], o.at[pl.ds(c * RCHUNK, RCHUNK)], semB[b])

        def wait_out(k, b):
            c = wid + k * NW

            @pl.when(c < NCHUNKS)
            def _():
                pltpu.make_async_copy(o.at[pl.ds(0, RCHUNK)], bufB[b], semB[b]).wait()

        def repack(b):
            @plsc.parallel_loop(0, RCHUNK, 1, unroll=4)
            def rp(j):
                for h in range(2):
                    for cc in range(EMBED // LANES):
                        bufB[b][j, pl.ds(h * EMBED + cc * LANES, LANES)] = (
                            bufA[b][2 * j + h, pl.ds(cc * LANES, LANES)])

        issue_in(0, 0)
        issue_in(1, 1)

        def pair_body(k2, carry):
            for b in range(2):
                k = 2 * k2 + b
                wait_in(k, b)
                @pl.when(k2 > 0)
                def _():
                    wait_out(k - 2, b)
                c = wid + k * NW

                @pl.when(c < NCHUNKS)
                def _():
                    repack(b)
                issue_out(k, b)
                issue_in(k + 2, b)
            return carry

        lax.fori_loop(0, KMAX // 2, pair_body, 0)
        wait_out(KMAX - 2, 0)
        wait_out(KMAX - 1, 1)


_relayout = functools.partial(
    pl.kernel,
    mesh=plsc.VectorSubcoreMesh(core_axis_name="c", subcore_axis_name="s"),
    out_type=tuple(
        jax.ShapeDtypeStruct((VOCAB // 2, 2 * EMBED), jnp.float32)
        for _ in range(NF)),
    scratch_types=[
        pltpu.VMEM((2 * RCHUNK, EMBED), jnp.float32),
        pltpu.VMEM((2 * RCHUNK, EMBED), jnp.float32),
        pltpu.VMEM((RCHUNK, 2 * EMBED), jnp.float32),
        pltpu.VMEM((RCHUNK, 2 * EMBED), jnp.float32),
        pltpu.SemaphoreType.DMA,
        pltpu.SemaphoreType.DMA,
        pltpu.SemaphoreType.DMA,
        pltpu.SemaphoreType.DMA,
    ],
    compiler_params=pltpu.CompilerParams(needs_layout_passes=True),
)(_relayout_body)


def _gather_body(i0, i1, i2, i3, i4, i5, i6, i7,
                 w0, w1, w2, w3, w4, w5, w6, w7,
                 out, idx_v, idxj_v, buf, outb, sem):
    idxs = [i0, i1, i2, i3, i4, i5, i6, i7]
    tables = [w0, w1, w2, w3, w4, w5, w6, w7]
    wid = lax.axis_index("s") * NC + lax.axis_index("c")
    base = wid * BPW

    for f in range(NF):
        pltpu.sync_copy(idxs[f].at[pl.ds(base, BPW)], idx_v.at[f])

    @plsc.parallel_loop(0, BPW // LANES, 1, unroll=4)
    def shift(i):
        for f in range(NF):
            v = idx_v[f, pl.ds(i * LANES, LANES)]
            idxj_v[f, pl.ds(i * LANES, LANES)] = v >> 1

    def round_body(r, carry):
        cps = [
            pltpu.async_copy(
                tables[f].at[idxj_v.at[f, pl.ds(r * CHUNK, CHUNK)]],
                buf.at[f], sem)
            for f in range(NF)
        ]
        for cp in cps:
            cp.wait()

        @plsc.parallel_loop(0, GROUPS, 1)
        def sum_group(g):
            vecs = [idx_v[f, pl.ds(r * CHUNK + g * LANES, LANES)]
                    for f in range(NF)]
            for jj in range(LANES):
                i = g * LANES + jj
                starts = [(vecs[f][jj] & 1) * EMBED for f in range(NF)]
                for c in range(EMBED // LANES):
                    acc = buf[0, i, pl.ds(starts[0] + c * LANES, LANES)]
                    for f in range(1, NF):
                        acc = acc + buf[f, i, pl.ds(starts[f] + c * LANES, LANES)]
                    outb[i, pl.ds(c * LANES, LANES)] = acc
        pltpu.sync_copy(outb, out.at[pl.ds(base + r * CHUNK, CHUNK)])
        return carry

    lax.fori_loop(0, ROUNDS, round_body, 0)


_poly_gather = functools.partial(
    pl.kernel,
    mesh=plsc.VectorSubcoreMesh(core_axis_name="c", subcore_axis_name="s"),
    out_type=jax.ShapeDtypeStruct((BATCH, EMBED), jnp.float32),
    scratch_types=[
        pltpu.VMEM((NF, BPW), jnp.int32),
        pltpu.VMEM((NF, BPW), jnp.int32),
        pltpu.VMEM((NF, CHUNK, 2 * EMBED), jnp.float32),
        pltpu.VMEM((CHUNK, EMBED), jnp.float32),
        pltpu.SemaphoreType.DMA,
    ],
)(_gather_body)


@jax.jit
def kernel(idx_0, idx_1, idx_2, idx_3, idx_4, idx_5, idx_6, idx_7,
           W_0, W_1, W_2, W_3, W_4, W_5, W_6, W_7):
    packed = _relayout(W_0, W_1, W_2, W_3, W_4, W_5, W_6, W_7)
    return _poly_gather(idx_0, idx_1, idx_2, idx_3, idx_4, idx_5, idx_6, idx_7,
                        *packed)


# speedup vs baseline: 1.0447x; 1.0447x over previous
import functools

import jax
import jax.numpy as jnp
from jax import lax
from jax.experimental import pallas as pl
from jax.experimental.pallas import tpu as pltpu
from jax.experimental.pallas import tpu_sc as plsc

NF = 8
VOCAB = 100000
BATCH = 16384
EMBED = 64
LANES = 16
TROWS = 8

NC = 2
NS = 16
NW = NC * NS
BPW = BATCH // NW
CHUNK = 16
ROUNDS = BPW // CHUNK


def _body(i0, i1, i2, i3, i4, i5, i6, i7,
          w0, w1, w2, w3, w4, w5, w6, w7,
          out, idx_v, slabs, outb, sem):
    idxs = [i0, i1, i2, i3, i4, i5, i6, i7]
    tables = [w0, w1, w2, w3, w4, w5, w6, w7]
    wid = lax.axis_index("s") * NC + lax.axis_index("c")
    base = wid * BPW

    for f in range(NF):
        pltpu.sync_copy(idxs[f].at[pl.ds(base, BPW)], idx_v.at[f])

    HALF = CHUNK // 2

    def round_body(r, carry):
        vecs = [idx_v[f, pl.ds(r * CHUNK, CHUNK)] for f in range(NF)]
        lane = lax.iota(jnp.int32, LANES)
        for h in range(2):
            cps = []
            for f in range(NF):
                tile0 = vecs[f] & ~(TROWS - 1)
                for jj in range(HALF):
                    j = h * HALF + jj
                    start = pl.multiple_of(tile0[j], TROWS)
                    cps.append(pltpu.async_copy(
                        tables[f].at[pl.ds(start, TROWS), :],
                        slabs.at[f, pl.ds(jj * TROWS, TROWS), :], sem))
            for cp in cps:
                cp.wait()

            for jj in range(HALF):
                j = h * HALF + jj
                rows = [jnp.zeros((LANES,), jnp.int32)
                        + (jj * TROWS + (vecs[f][j] & (TROWS - 1)))
                        for f in range(NF)]
                for c in range(EMBED // LANES):
                    cols = lane + (c * LANES)
                    acc = plsc.load_gather(slabs.at[0], [rows[0], cols])
                    for f in range(1, NF):
                        acc = acc + plsc.load_gather(slabs.at[f], [rows[f], cols])
                    outb[j, pl.ds(c * LANES, LANES)] = acc

        pltpu.sync_copy(outb, out.at[pl.ds(base + r * CHUNK, CHUNK)])
        return carry

    lax.fori_loop(0, ROUNDS, round_body, 0)


_poly_embed = functools.partial(
    pl.kernel,
    mesh=plsc.VectorSubcoreMesh(core_axis_name="c", subcore_axis_name="s"),
    out_type=jax.ShapeDtypeStruct((BATCH, EMBED), jnp.float32),
    scratch_types=[
        pltpu.VMEM((NF, BPW), jnp.int32),
        pltpu.VMEM((NF, (CHUNK // 2) * TROWS, EMBED), jnp.float32),
        pltpu.VMEM((CHUNK, EMBED), jnp.float32),
        pltpu.SemaphoreType.DMA,
    ],
    compiler_params=pltpu.CompilerParams(needs_layout_passes=False),
)(_body)


@jax.jit
def kernel(idx_0, idx_1, idx_2, idx_3, idx_4, idx_5, idx_6, idx_7,
           W_0, W_1, W_2, W_3, W_4, W_5, W_6, W_7):
    return _poly_embed(idx_0, idx_1, idx_2, idx_3, idx_4, idx_5, idx_6, idx_7,
                       W_0, W_1, W_2, W_3, W_4, W_5, W_6, W_7)
